# baseline (speedup 1.0000x reference)
import functools

import jax
import jax.numpy as jnp
from jax import lax
from jax.experimental import pallas as pl
from jax.experimental.pallas import tpu as pltpu
from jax.experimental.pallas import tpu_sc as plsc

D_MODEL = 4096
NUM_CORES = 2
NUM_SUBCORES = 16
NUM_WORKERS = NUM_CORES * NUM_SUBCORES
TOTAL_IDS = 8192
SC_ROWS = 6144
TC_ROWS = TOTAL_IDS - SC_ROWS
CHUNK = 4
NBUF = 4
IDS_PER_WORKER = SC_ROWS // NUM_WORKERS
NUM_CHUNKS = IDS_PER_WORKER // CHUNK


def _make_emb_kernel():
    mesh = plsc.VectorSubcoreMesh(core_axis_name="c", subcore_axis_name="s")

    @functools.partial(
        pl.kernel,
        mesh=mesh,
        out_type=jax.ShapeDtypeStruct((TOTAL_IDS, D_MODEL), jnp.float32),
        scratch_types=[
            pltpu.VMEM((NUM_CHUNKS, CHUNK), jnp.int32),
            pltpu.VMEM((NBUF, CHUNK, D_MODEL), jnp.float32),
        ] + [pltpu.SemaphoreType.DMA] * (2 * NBUF),
    )
    def emb(idx_hbm, table_hbm, out_hbm, idx_v, rows_v, *sems):
        gsems = sems[:NBUF]
        ssems = sems[NBUF:]
        wid = lax.axis_index("s") * NUM_CORES + lax.axis_index("c")
        base = wid * IDS_PER_WORKER
        pltpu.sync_copy(idx_hbm.at[wid], idx_v)

        def start_gather(i, b):
            pltpu.async_copy(table_hbm.at[idx_v.at[i]], rows_v.at[b], gsems[b])

        def wait_gather(i, b):
            pltpu.make_async_copy(
                table_hbm.at[idx_v.at[i]], rows_v.at[b], gsems[b]
            ).wait()

        def start_store(i, b):
            pltpu.async_copy(
                rows_v.at[b], out_hbm.at[pl.ds(base + i * CHUNK, CHUNK)], ssems[b]
            )

        def wait_store(i, b):
            pltpu.make_async_copy(
                rows_v.at[b], out_hbm.at[pl.ds(base + i * CHUNK, CHUNK)], ssems[b]
            ).wait()

        for b in range(NBUF):
            start_gather(b, b)

        def group(g, carry):
            for b in range(NBUF):
                i = g * NBUF + b
                wait_gather(i, b)
                start_store(i, b)
            for b in range(NBUF):
                i = g * NBUF + b
                wait_store(i, b)

                @pl.when(i + NBUF < NUM_CHUNKS)
                def _():
                    start_gather(i + NBUF, b)

            return carry

        lax.fori_loop(0, NUM_CHUNKS // NBUF, group, 0)

    return emb


def _make_tc_gather(n_rows):
    def body(idx_ref, row_ref, out_ref):
        out_ref[...] = row_ref[...]

    return pl.pallas_call(
        body,
        grid_spec=pltpu.PrefetchScalarGridSpec(
            num_scalar_prefetch=1,
            grid=(n_rows,),
            in_specs=[
                pl.BlockSpec(
                    (1, 1, D_MODEL), lambda i, idx_ref: (idx_ref[i], 0, 0)
                )
            ],
            out_specs=pl.BlockSpec(
                (1, 1, D_MODEL), lambda i, idx_ref: (i, 0, 0)
            ),
        ),
        out_shape=jax.ShapeDtypeStruct((n_rows, 1, D_MODEL), jnp.float32),
    )


_emb = _make_emb_kernel()
_tc_gather = _make_tc_gather(TC_ROWS)


def kernel(input_ids, wte):
    input_shape = input_ids.shape
    flat = input_ids.reshape(-1).astype(jnp.int32)
    sc_idx = flat[:SC_ROWS].reshape(NUM_WORKERS, NUM_CHUNKS, CHUNK)
    tc_idx = flat[SC_ROWS:]
    out = _emb(sc_idx, wte)
    tc_part = _tc_gather(tc_idx, wte.reshape(-1, 1, D_MODEL))
    out = lax.dynamic_update_slice(
        out, tc_part.reshape(TC_ROWS, D_MODEL), (SC_ROWS, 0)
    )
    return out.reshape((-1, input_shape[-1], D_MODEL))


# speedup vs baseline: 16.8580x; 16.8580x over previous
import functools

import jax
import jax.numpy as jnp
from jax import lax
from jax.experimental import pallas as pl
from jax.experimental.pallas import tpu as pltpu
from jax.experimental.pallas import tpu_sc as plsc

D_MODEL = 4096
NUM_CORES = 2
NUM_SUBCORES = 16
NUM_WORKERS = NUM_CORES * NUM_SUBCORES
TOTAL_IDS = 8192
IDS_PER_WORKER = TOTAL_IDS // NUM_WORKERS
CHUNK = 4
NUM_CHUNKS = IDS_PER_WORKER // CHUNK


NBUF = 4


def _make_emb_kernel():
    mesh = plsc.VectorSubcoreMesh(core_axis_name="c", subcore_axis_name="s")

    @functools.partial(
        pl.kernel,
        mesh=mesh,
        out_type=jax.ShapeDtypeStruct((TOTAL_IDS, D_MODEL), jnp.float32),
        scratch_types=[
            pltpu.VMEM((NUM_CHUNKS, CHUNK), jnp.int32),
            pltpu.VMEM((NBUF, CHUNK, D_MODEL), jnp.float32),
        ] + [pltpu.SemaphoreType.DMA] * (2 * NBUF),
    )
    def emb(idx_hbm, table_hbm, out_hbm, idx_v, rows_v, *sems):
        gsems = sems[:NBUF]
        ssems = sems[NBUF:]
        wid = lax.axis_index("c") * NUM_SUBCORES + lax.axis_index("s")
        base = wid * IDS_PER_WORKER
        pltpu.sync_copy(idx_hbm.at[wid], idx_v)

        def start_gather(i, b):
            pltpu.async_copy(table_hbm.at[idx_v.at[i]], rows_v.at[b], gsems[b])

        def wait_gather(i, b):
            pltpu.make_async_copy(
                table_hbm.at[idx_v.at[i]], rows_v.at[b], gsems[b]
            ).wait()

        def start_store(i, b):
            pltpu.async_copy(
                rows_v.at[b], out_hbm.at[pl.ds(base + i * CHUNK, CHUNK)], ssems[b]
            )

        def wait_store(i, b):
            pltpu.make_async_copy(
                rows_v.at[b], out_hbm.at[pl.ds(base + i * CHUNK, CHUNK)], ssems[b]
            ).wait()

        for b in range(NBUF):
            start_gather(b, b)

        def group(g, carry):
            for b in range(NBUF):
                i = g * NBUF + b
                wait_gather(i, b)
                start_store(i, b)
            for b in range(NBUF):
                i = g * NBUF + b
                wait_store(i, b)

                @pl.when(i + NBUF < NUM_CHUNKS)
                def _():
                    start_gather(i + NBUF, b)

            return carry

        lax.fori_loop(0, NUM_CHUNKS // NBUF, group, 0)

    return emb


_emb = _make_emb_kernel()


def kernel(input_ids, wte):
    input_shape = input_ids.shape
    flat = input_ids.reshape(-1).astype(jnp.int32)
    idx3 = flat.reshape(NUM_WORKERS, NUM_CHUNKS, CHUNK)
    out = _emb(idx3, wte)
    return out.reshape((-1, input_shape[-1], D_MODEL))
